# Initial kernel scaffold; baseline (speedup 1.0000x reference)
import jax
import jax.numpy as jnp
from jax.experimental import pallas as pl


def kernel(x, neighbor_id_lstlst, weights, bias):
    raise NotImplementedError("write your pallas kernel here")


# speedup vs baseline: 2.0948x; 2.0948x over previous
import functools

import jax
import jax.numpy as jnp
from jax import lax
from jax.experimental import pallas as pl
from jax.experimental.pallas import tpu as pltpu
from jax.experimental.pallas import tpu_sc as plsc

B = 4
N = 50000
K = 16
CIN = 3
COUT = 16

NW = 32
ROWS = N * K
ROWS_PER_W = ROWS // NW
CHUNK = 5000

NB = 400


def _sc_gather(table, idx_flat):
    mesh = plsc.VectorSubcoreMesh(core_axis_name="c", subcore_axis_name="s")

    @functools.partial(
        pl.kernel,
        out_type=jax.ShapeDtypeStruct((ROWS, 16), jnp.float32),
        mesh=mesh,
        compiler_params=pltpu.CompilerParams(use_tc_tiling_on_sc=False),
        scratch_types=[
            pltpu.VMEM((CHUNK,), jnp.int32),
            pltpu.VMEM((CHUNK, 16), jnp.float32),
            pltpu.SemaphoreType.DMA,
        ],
    )
    def k(table_hbm, idx_hbm, out_hbm, idx_v, rows_v, sem):
        wid = lax.axis_index("s") * 2 + lax.axis_index("c")
        base = wid * ROWS_PER_W

        def body(i, carry):
            off = base + i * CHUNK
            pltpu.sync_copy(idx_hbm.at[pl.ds(off, CHUNK)], idx_v)
            pltpu.async_copy(table_hbm.at[idx_v], rows_v, sem).wait()
            pltpu.sync_copy(rows_v, out_hbm.at[pl.ds(off, CHUNK)])
            return carry

        lax.fori_loop(0, ROWS_PER_W // CHUNK, body, 0)

    return k(table, idx_flat)


def _tc_body(g_ref, w_ref, b_ref, o_ref):
    g = g_ref[...]
    w = w_ref[...]
    for b in range(B):
        acc = None
        for k in range(K):
            gk = g[:, 16 * k + 4 * b:16 * k + 4 * b + 3]
            gt = jnp.concatenate([gk] * COUT, axis=1)
            p = gt * w[:, 48 * k:48 * k + 48]
            acc = p if acc is None else acc + p
        ob = acc.reshape(NB, COUT, CIN).sum(axis=-1) + b_ref[...]
        o_ref[b] = jnp.where(ob > 0, ob, jnp.exp(ob) - 1.0)


def _tc_reduce(gathered, weights_flat, bias):
    grid = N // NB
    return pl.pallas_call(
        _tc_body,
        grid=(grid,),
        in_specs=[
            pl.BlockSpec((NB, 256), lambda i: (i, 0)),
            pl.BlockSpec((NB, 768), lambda i: (i, 0)),
            pl.BlockSpec((NB, COUT), lambda i: (i, 0)),
        ],
        out_specs=pl.BlockSpec((B, NB, COUT), lambda i: (0, i, 0)),
        out_shape=jax.ShapeDtypeStruct((B, N, COUT), jnp.float32),
    )(gathered, weights_flat, bias)


def kernel(x, neighbor_id_lstlst, weights, bias):
    xt = jnp.transpose(x, (1, 0, 2))
    xt = jnp.pad(xt, ((0, 0), (0, 0), (0, 1)))
    table = xt.reshape(N, 16)
    idx_flat = neighbor_id_lstlst.astype(jnp.int32).reshape(ROWS)

    gathered = _sc_gather(table, idx_flat)
    g2 = gathered.reshape(N, 256)
    wf = weights.reshape(N, K * COUT * CIN)
    return _tc_reduce(g2, wf, bias)
